# baseline (speedup 1.0000x reference)
import jax
import jax.numpy as jnp
from jax.experimental import pallas as pl
from jax.experimental.pallas import tpu as pltpu

_EMB = 64
_NEMB = 100
_LEAD = 2
_BLK = _LEAD * 32 * 32


def _body(x_ref, w2x_ref, wt1_ref, wt2_ref, wt3_ref, c_ref, w2_ref,
          idx_ref, q_ref):
    xb = x_ref[...].reshape(_BLK, _EMB)
    xw2 = jax.lax.dot_general(
        xb, w2x_ref[...], dimension_numbers=(((1,), (0,)), ((), ())),
        preferred_element_type=jnp.float32)
    neg = (xw2 - c_ref[...]) - w2_ref[...]
    m = jnp.max(neg, axis=1, keepdims=True)
    colsf = jax.lax.broadcasted_iota(
        jnp.int32, neg.shape, 1).astype(jnp.float32)
    idxf = jnp.min(jnp.where(neg == m, colsf, jnp.float32(_NEMB)),
                   axis=1, keepdims=True)
    idx_ref[...] = idxf.astype(jnp.int32).reshape(_LEAD, 32, 32)
    onehot = (colsf == idxf).astype(jnp.bfloat16)
    dn = (((1,), (0,)), ((), ()))
    q = ((jax.lax.dot_general(onehot, wt1_ref[...], dn,
                              preferred_element_type=jnp.float32)
          + jax.lax.dot_general(onehot, wt2_ref[...], dn,
                                preferred_element_type=jnp.float32))
         + jax.lax.dot_general(onehot, wt3_ref[...], dn,
                               preferred_element_type=jnp.float32))
    q_ref[...] = q.reshape(_LEAD, 32, 32, _EMB)


def kernel(x, w):
    lead = x.shape[0]
    nb = lead // _LEAD
    w2x = 2.0 * w
    wt = w.T
    def _tr(v):
        b = jax.lax.bitcast_convert_type(v, jnp.uint32)
        return jax.lax.bitcast_convert_type(b & jnp.uint32(0xFFFF0000),
                                            jnp.float32)
    p1 = _tr(wt)
    r1 = wt - p1
    p2 = _tr(r1)
    p3 = r1 - p2
    wt1 = p1.astype(jnp.bfloat16)
    wt2 = p2.astype(jnp.bfloat16)
    wt3 = p3.astype(jnp.bfloat16)
    c = jnp.sum(x.reshape(-1, _EMB) ** 2, axis=1, keepdims=True)
    w2 = jnp.sum(w ** 2, axis=0, keepdims=True)
    idx, q = pl.pallas_call(
        _body,
        grid=(nb,),
        in_specs=[
            pl.BlockSpec((_LEAD, 32, 32, _EMB), lambda i: (i, 0, 0, 0)),
            pl.BlockSpec((_EMB, _NEMB), lambda i: (0, 0)),
            pl.BlockSpec((_NEMB, _EMB), lambda i: (0, 0)),
            pl.BlockSpec((_NEMB, _EMB), lambda i: (0, 0)),
            pl.BlockSpec((_NEMB, _EMB), lambda i: (0, 0)),
            pl.BlockSpec((_BLK, 1), lambda i: (i, 0)),
            pl.BlockSpec((1, _NEMB), lambda i: (0, 0)),
        ],
        out_specs=[
            pl.BlockSpec((_LEAD, 32, 32), lambda i: (i, 0, 0)),
            pl.BlockSpec((_LEAD, 32, 32, _EMB), lambda i: (i, 0, 0, 0)),
        ],
        out_shape=[
            jax.ShapeDtypeStruct(x.shape[:-1], jnp.int32),
            jax.ShapeDtypeStruct(x.shape, jnp.float32),
        ],
        compiler_params=pltpu.CompilerParams(
            dimension_semantics=("parallel",)),
    )(x, w2x, wt1, wt2, wt3, c, w2)
    return q, idx


# speedup vs baseline: 1.1134x; 1.1134x over previous
import jax
import jax.numpy as jnp
from jax.experimental import pallas as pl
from jax.experimental.pallas import tpu as pltpu

_EMB = 64
_NEMB = 100
_LEAD = 2
_BLK = _LEAD * 32 * 32


def _body(x_ref, w2x_ref, wt1_ref, wt2_ref, wt3_ref, c_ref, w2_ref,
          idx_ref, q_ref):
    xb = x_ref[...].reshape(_BLK, _EMB)
    xw2 = jax.lax.dot_general(
        xb, w2x_ref[...], dimension_numbers=(((1,), (0,)), ((), ())),
        preferred_element_type=jnp.float32)
    cb = c_ref[0].reshape(_BLK, 1)
    neg = (xw2 - cb) - w2_ref[...]
    m = jnp.max(neg, axis=1, keepdims=True)
    colsf = jax.lax.broadcasted_iota(
        jnp.int32, neg.shape, 1).astype(jnp.float32)
    idxf = jnp.min(jnp.where(neg == m, colsf, jnp.float32(_NEMB)),
                   axis=1, keepdims=True)
    idx_ref[...] = idxf.astype(jnp.int32).reshape(_LEAD, 32, 32)
    onehot = (colsf == idxf).astype(jnp.bfloat16)
    dn = (((1,), (0,)), ((), ()))
    q = ((jax.lax.dot_general(onehot, wt1_ref[...], dn,
                              preferred_element_type=jnp.float32)
          + jax.lax.dot_general(onehot, wt2_ref[...], dn,
                                preferred_element_type=jnp.float32))
         + jax.lax.dot_general(onehot, wt3_ref[...], dn,
                               preferred_element_type=jnp.float32))
    q_ref[...] = q.reshape(_LEAD, 32, 32, _EMB)


def kernel(x, w):
    lead = x.shape[0]
    nb = lead // _LEAD
    w2x = 2.0 * w
    wt = w.T
    def _tr(v):
        b = jax.lax.bitcast_convert_type(v, jnp.uint32)
        return jax.lax.bitcast_convert_type(b & jnp.uint32(0xFFFF0000),
                                            jnp.float32)
    p1 = _tr(wt)
    r1 = wt - p1
    p2 = _tr(r1)
    p3 = r1 - p2
    wt1 = p1.astype(jnp.bfloat16)
    wt2 = p2.astype(jnp.bfloat16)
    wt3 = p3.astype(jnp.bfloat16)
    c = jnp.sum(x.reshape(-1, _EMB) ** 2, axis=1).reshape(nb, 1, _BLK)
    w2 = jnp.sum(w ** 2, axis=0, keepdims=True)
    idx, q = pl.pallas_call(
        _body,
        grid=(nb,),
        in_specs=[
            pl.BlockSpec((_LEAD, 32, 32, _EMB), lambda i: (i, 0, 0, 0)),
            pl.BlockSpec((_EMB, _NEMB), lambda i: (0, 0)),
            pl.BlockSpec((_NEMB, _EMB), lambda i: (0, 0)),
            pl.BlockSpec((_NEMB, _EMB), lambda i: (0, 0)),
            pl.BlockSpec((_NEMB, _EMB), lambda i: (0, 0)),
            pl.BlockSpec((1, 1, _BLK), lambda i: (i, 0, 0)),
            pl.BlockSpec((1, _NEMB), lambda i: (0, 0)),
        ],
        out_specs=[
            pl.BlockSpec((_LEAD, 32, 32), lambda i: (i, 0, 0)),
            pl.BlockSpec((_LEAD, 32, 32, _EMB), lambda i: (i, 0, 0, 0)),
        ],
        out_shape=[
            jax.ShapeDtypeStruct(x.shape[:-1], jnp.int32),
            jax.ShapeDtypeStruct(x.shape, jnp.float32),
        ],
        compiler_params=pltpu.CompilerParams(
            dimension_semantics=("parallel",)),
    )(x, w2x, wt1, wt2, wt3, c, w2)
    return q, idx


# speedup vs baseline: 1.2955x; 1.1635x over previous
import jax
import jax.numpy as jnp
from jax.experimental import pallas as pl
from jax.experimental.pallas import tpu as pltpu

_EMB = 64
_NEMB = 100
_LEAD = 4
_BLK = _LEAD * 32 * 32


def _body(x_ref, w2x_ref, wt1_ref, wt2_ref, wt3_ref, c_ref, w2_ref,
          idx_ref, q_ref):
    xb = x_ref[...].reshape(_BLK, _EMB)
    xw2 = jax.lax.dot_general(
        xb, w2x_ref[...], dimension_numbers=(((1,), (0,)), ((), ())),
        preferred_element_type=jnp.float32)
    cb = c_ref[0].reshape(_BLK, 1)
    neg = (xw2 - cb) - w2_ref[...]
    m = jnp.max(neg, axis=1, keepdims=True)
    colsf = jax.lax.broadcasted_iota(
        jnp.int32, neg.shape, 1).astype(jnp.float32)
    idxf = jnp.min(jnp.where(neg == m, colsf, jnp.float32(_NEMB)),
                   axis=1, keepdims=True)
    idx_ref[...] = idxf.astype(jnp.int32).reshape(_LEAD, 32, 32)
    onehot = (colsf == idxf).astype(jnp.bfloat16)
    dn = (((1,), (0,)), ((), ()))
    q = ((jax.lax.dot_general(onehot, wt1_ref[...], dn,
                              preferred_element_type=jnp.float32)
          + jax.lax.dot_general(onehot, wt2_ref[...], dn,
                                preferred_element_type=jnp.float32))
         + jax.lax.dot_general(onehot, wt3_ref[...], dn,
                               preferred_element_type=jnp.float32))
    q_ref[...] = q.reshape(_LEAD, 32, 32, _EMB)


def kernel(x, w):
    lead = x.shape[0]
    nb = lead // _LEAD
    w2x = 2.0 * w
    wt = w.T
    def _tr(v):
        b = jax.lax.bitcast_convert_type(v, jnp.uint32)
        return jax.lax.bitcast_convert_type(b & jnp.uint32(0xFFFF0000),
                                            jnp.float32)
    p1 = _tr(wt)
    r1 = wt - p1
    p2 = _tr(r1)
    p3 = r1 - p2
    wt1 = p1.astype(jnp.bfloat16)
    wt2 = p2.astype(jnp.bfloat16)
    wt3 = p3.astype(jnp.bfloat16)
    c = jnp.sum(x.reshape(-1, _EMB) ** 2, axis=1).reshape(nb, 1, _BLK)
    w2 = jnp.sum(w ** 2, axis=0, keepdims=True)
    idx, q = pl.pallas_call(
        _body,
        grid=(nb,),
        in_specs=[
            pl.BlockSpec((_LEAD, 32, 32, _EMB), lambda i: (i, 0, 0, 0)),
            pl.BlockSpec((_EMB, _NEMB), lambda i: (0, 0)),
            pl.BlockSpec((_NEMB, _EMB), lambda i: (0, 0)),
            pl.BlockSpec((_NEMB, _EMB), lambda i: (0, 0)),
            pl.BlockSpec((_NEMB, _EMB), lambda i: (0, 0)),
            pl.BlockSpec((1, 1, _BLK), lambda i: (i, 0, 0)),
            pl.BlockSpec((1, _NEMB), lambda i: (0, 0)),
        ],
        out_specs=[
            pl.BlockSpec((_LEAD, 32, 32), lambda i: (i, 0, 0)),
            pl.BlockSpec((_LEAD, 32, 32, _EMB), lambda i: (i, 0, 0, 0)),
        ],
        out_shape=[
            jax.ShapeDtypeStruct(x.shape[:-1], jnp.int32),
            jax.ShapeDtypeStruct(x.shape, jnp.float32),
        ],
        compiler_params=pltpu.CompilerParams(
            dimension_semantics=("parallel",)),
    )(x, w2x, wt1, wt2, wt3, c, w2)
    return q, idx


# speedup vs baseline: 1.4130x; 1.0907x over previous
import jax
import jax.numpy as jnp
from jax.experimental import pallas as pl
from jax.experimental.pallas import tpu as pltpu

_EMB = 64
_NEMB = 100
_LEAD = 8
_BLK = _LEAD * 32 * 32


def _body(x_ref, w2x_ref, wt1_ref, wt2_ref, wt3_ref, c_ref, w2_ref,
          idx_ref, q_ref):
    xb = x_ref[...].reshape(_BLK, _EMB)
    xw2 = jax.lax.dot_general(
        xb, w2x_ref[...], dimension_numbers=(((1,), (0,)), ((), ())),
        preferred_element_type=jnp.float32)
    cb = c_ref[0].reshape(_BLK, 1)
    neg = (xw2 - cb) - w2_ref[...]
    m = jnp.max(neg, axis=1, keepdims=True)
    colsf = jax.lax.broadcasted_iota(
        jnp.int32, neg.shape, 1).astype(jnp.float32)
    idxf = jnp.min(jnp.where(neg == m, colsf, jnp.float32(_NEMB)),
                   axis=1, keepdims=True)
    idx_ref[...] = idxf.astype(jnp.int32).reshape(_LEAD, 32, 32)
    onehot = (colsf == idxf).astype(jnp.bfloat16)
    dn = (((1,), (0,)), ((), ()))
    q = ((jax.lax.dot_general(onehot, wt1_ref[...], dn,
                              preferred_element_type=jnp.float32)
          + jax.lax.dot_general(onehot, wt2_ref[...], dn,
                                preferred_element_type=jnp.float32))
         + jax.lax.dot_general(onehot, wt3_ref[...], dn,
                               preferred_element_type=jnp.float32))
    q_ref[...] = q.reshape(_LEAD, 32, 32, _EMB)


def kernel(x, w):
    lead = x.shape[0]
    nb = lead // _LEAD
    w2x = 2.0 * w
    wt = w.T
    def _tr(v):
        b = jax.lax.bitcast_convert_type(v, jnp.uint32)
        return jax.lax.bitcast_convert_type(b & jnp.uint32(0xFFFF0000),
                                            jnp.float32)
    p1 = _tr(wt)
    r1 = wt - p1
    p2 = _tr(r1)
    p3 = r1 - p2
    wt1 = p1.astype(jnp.bfloat16)
    wt2 = p2.astype(jnp.bfloat16)
    wt3 = p3.astype(jnp.bfloat16)
    c = jnp.sum(x.reshape(-1, _EMB) ** 2, axis=1).reshape(nb, 1, _BLK)
    w2 = jnp.sum(w ** 2, axis=0, keepdims=True)
    idx, q = pl.pallas_call(
        _body,
        grid=(nb,),
        in_specs=[
            pl.BlockSpec((_LEAD, 32, 32, _EMB), lambda i: (i, 0, 0, 0)),
            pl.BlockSpec((_EMB, _NEMB), lambda i: (0, 0)),
            pl.BlockSpec((_NEMB, _EMB), lambda i: (0, 0)),
            pl.BlockSpec((_NEMB, _EMB), lambda i: (0, 0)),
            pl.BlockSpec((_NEMB, _EMB), lambda i: (0, 0)),
            pl.BlockSpec((1, 1, _BLK), lambda i: (i, 0, 0)),
            pl.BlockSpec((1, _NEMB), lambda i: (0, 0)),
        ],
        out_specs=[
            pl.BlockSpec((_LEAD, 32, 32), lambda i: (i, 0, 0)),
            pl.BlockSpec((_LEAD, 32, 32, _EMB), lambda i: (i, 0, 0, 0)),
        ],
        out_shape=[
            jax.ShapeDtypeStruct(x.shape[:-1], jnp.int32),
            jax.ShapeDtypeStruct(x.shape, jnp.float32),
        ],
        compiler_params=pltpu.CompilerParams(
            dimension_semantics=("parallel",)),
    )(x, w2x, wt1, wt2, wt3, c, w2)
    return q, idx


# speedup vs baseline: 1.4675x; 1.0386x over previous
import jax
import jax.numpy as jnp
from jax.experimental import pallas as pl
from jax.experimental.pallas import tpu as pltpu

_EMB = 64
_NEMB = 100
_LEAD = 16
_BLK = _LEAD * 32 * 32


def _body(x_ref, w2x_ref, wt1_ref, wt2_ref, wt3_ref, c_ref, w2_ref,
          idx_ref, q_ref):
    xb = x_ref[...].reshape(_BLK, _EMB)
    xw2 = jax.lax.dot_general(
        xb, w2x_ref[...], dimension_numbers=(((1,), (0,)), ((), ())),
        preferred_element_type=jnp.float32)
    cb = c_ref[0].reshape(_BLK, 1)
    neg = (xw2 - cb) - w2_ref[...]
    m = jnp.max(neg, axis=1, keepdims=True)
    colsf = jax.lax.broadcasted_iota(
        jnp.int32, neg.shape, 1).astype(jnp.float32)
    idxf = jnp.min(jnp.where(neg == m, colsf, jnp.float32(_NEMB)),
                   axis=1, keepdims=True)
    idx_ref[...] = idxf.astype(jnp.int32).reshape(_LEAD, 32, 32)
    onehot = (colsf == idxf).astype(jnp.bfloat16)
    dn = (((1,), (0,)), ((), ()))
    q = ((jax.lax.dot_general(onehot, wt1_ref[...], dn,
                              preferred_element_type=jnp.float32)
          + jax.lax.dot_general(onehot, wt2_ref[...], dn,
                                preferred_element_type=jnp.float32))
         + jax.lax.dot_general(onehot, wt3_ref[...], dn,
                               preferred_element_type=jnp.float32))
    q_ref[...] = q.reshape(_LEAD, 32, 32, _EMB)


def kernel(x, w):
    lead = x.shape[0]
    nb = lead // _LEAD
    w2x = 2.0 * w
    wt = w.T
    def _tr(v):
        b = jax.lax.bitcast_convert_type(v, jnp.uint32)
        return jax.lax.bitcast_convert_type(b & jnp.uint32(0xFFFF0000),
                                            jnp.float32)
    p1 = _tr(wt)
    r1 = wt - p1
    p2 = _tr(r1)
    p3 = r1 - p2
    wt1 = p1.astype(jnp.bfloat16)
    wt2 = p2.astype(jnp.bfloat16)
    wt3 = p3.astype(jnp.bfloat16)
    c = jnp.sum(x.reshape(-1, _EMB) ** 2, axis=1).reshape(nb, 1, _BLK)
    w2 = jnp.sum(w ** 2, axis=0, keepdims=True)
    idx, q = pl.pallas_call(
        _body,
        grid=(nb,),
        in_specs=[
            pl.BlockSpec((_LEAD, 32, 32, _EMB), lambda i: (i, 0, 0, 0)),
            pl.BlockSpec((_EMB, _NEMB), lambda i: (0, 0)),
            pl.BlockSpec((_NEMB, _EMB), lambda i: (0, 0)),
            pl.BlockSpec((_NEMB, _EMB), lambda i: (0, 0)),
            pl.BlockSpec((_NEMB, _EMB), lambda i: (0, 0)),
            pl.BlockSpec((1, 1, _BLK), lambda i: (i, 0, 0)),
            pl.BlockSpec((1, _NEMB), lambda i: (0, 0)),
        ],
        out_specs=[
            pl.BlockSpec((_LEAD, 32, 32), lambda i: (i, 0, 0)),
            pl.BlockSpec((_LEAD, 32, 32, _EMB), lambda i: (i, 0, 0, 0)),
        ],
        out_shape=[
            jax.ShapeDtypeStruct(x.shape[:-1], jnp.int32),
            jax.ShapeDtypeStruct(x.shape, jnp.float32),
        ],
        compiler_params=pltpu.CompilerParams(
            dimension_semantics=("parallel",)),
    )(x, w2x, wt1, wt2, wt3, c, w2)
    return q, idx
